# Initial kernel scaffold; baseline (speedup 1.0000x reference)
import functools

import jax
import jax.numpy as jnp
from jax import lax
from jax.experimental import pallas as pl
from jax.experimental.pallas import tpu as pltpu
from jax.experimental.pallas import tpu_sc as plsc

N_NODES = 10000
N_PAD = 10240
N_EDGES = 320000
D_IN = 128
D_HID = 128
D_OUT = 64

NC, NS, LANES = 2, 16, 16
E_W = N_EDGES // (NC * NS)
E_C = 2 * E_W
CH = 80
NCH = E_W // CH
ROWS_W = N_PAD // NS


def _leaky(v):
    return jnp.where(v >= 0, v, 0.2 * v)


def _dense_tail(h, as_vec, ad_vec, h_ref, av_ref, bv_ref, sh_ref):
    h_ref[...] = h
    av = jnp.sum(h * as_vec[None, :], axis=1)
    bv = jnp.sum(h * ad_vec[None, :], axis=1)
    av_ref[...] = av
    bv_ref[...] = bv
    m = jnp.max(av) + jnp.max(bv)
    sh_ref[...] = jnp.full((LANES,), _leaky(m), jnp.float32)


def _tc1_body(x_ref, w_ref, as_ref, ad_ref, h_ref, av_ref, bv_ref, sh_ref):
    h = jnp.dot(x_ref[...], w_ref[...], preferred_element_type=jnp.float32)
    _dense_tail(h, as_ref[...], ad_ref[...], h_ref, av_ref, bv_ref, sh_ref)


def _tc2_body(p_ref, b_ref, w_ref, as_ref, ad_ref, h_ref, av_ref, bv_ref, sh_ref):
    xx = p_ref[0, :N_NODES, :] + p_ref[1, :N_NODES, :] + b_ref[...][None, :]
    xx = jnp.maximum(xx, 0.0)
    h = jnp.dot(xx, w_ref[...], preferred_element_type=jnp.float32)
    _dense_tail(h, as_ref[...], ad_ref[...], h_ref, av_ref, bv_ref, sh_ref)


def _tc3_body(p_ref, b_ref, o_ref):
    o_ref[...] = p_ref[0, :N_NODES, :] + p_ref[1, :N_NODES, :] + b_ref[...][None, :]


def _dense_out(d):
    return [
        jax.ShapeDtypeStruct((N_NODES, d), jnp.float32),
        jax.ShapeDtypeStruct((N_NODES,), jnp.float32),
        jax.ShapeDtypeStruct((N_NODES,), jnp.float32),
        jax.ShapeDtypeStruct((LANES,), jnp.float32),
    ]


_tc1 = pl.pallas_call(_tc1_body, out_shape=_dense_out(D_HID))
_tc2 = pl.pallas_call(_tc2_body, out_shape=_dense_out(D_OUT))
_tc3 = pl.pallas_call(
    _tc3_body, out_shape=jax.ShapeDtypeStruct((N_NODES, D_OUT), jnp.float32))


def _make_sc_edge(D):
    mesh = plsc.VectorSubcoreMesh(core_axis_name="c", subcore_axis_name="s")

    @functools.partial(
        pl.kernel,
        out_type=jax.ShapeDtypeStruct((NC, N_PAD, D), jnp.float32),
        mesh=mesh,
        scratch_types=[
            pltpu.VMEM((N_PAD,), jnp.float32),
            pltpu.VMEM((N_PAD,), jnp.float32),
            pltpu.VMEM((N_PAD,), jnp.float32),
            pltpu.VMEM((E_W,), jnp.int32),
            pltpu.VMEM((E_W,), jnp.int32),
            pltpu.VMEM((E_W,), jnp.float32),
            pltpu.VMEM((E_W,), jnp.float32),
            pltpu.VMEM((NCH, CH), jnp.int32),
            pltpu.VMEM((CH, D), jnp.float32),
            pltpu.VMEM((CH,), jnp.float32),
            pltpu.VMEM((ROWS_W,), jnp.float32),
            pltpu.VMEM((ROWS_W,), jnp.float32),
            pltpu.VMEM((LANES,), jnp.float32),
            pltpu.VMEM_SHARED((NS, N_PAD), jnp.float32),
            pltpu.VMEM_SHARED((N_PAD,), jnp.float32),
            pltpu.VMEM_SHARED((N_PAD, D), jnp.float32),
        ],
    )
    def sc_edge(src_hbm, dst_hbm, dst2_hbm, ew_hbm, as_hbm, ad_hbm, sh_hbm,
                h_hbm, out_hbm,
                as_v, ad_v, denom_v, src_v, dst_v, exv, ew_v, dst2_v, rows_v,
                coeff_v, acc_v, tmp_v, shift_v, denom_stage, denom_fin,
                out_acc):
        c = lax.axis_index("c")
        s = lax.axis_index("s")
        zero16 = jnp.zeros((LANES,), jnp.float32)

        pltpu.sync_copy(as_hbm, as_v.at[pl.ds(0, N_NODES)])
        pltpu.sync_copy(ad_hbm, ad_v.at[pl.ds(0, N_NODES)])
        pltpu.sync_copy(sh_hbm, shift_v)
        shift = shift_v[...]

        @pl.loop(0, N_PAD // LANES)
        def _(i):
            denom_v[pl.ds(i * LANES, LANES)] = zero16

        @pl.loop(0, CH)
        def _(e2):
            @pl.loop(0, D // LANES)
            def _(k2):
                rows_v[e2, pl.ds(k2 * LANES, LANES)] = zero16

        @pl.loop(0, ROWS_W // CH)
        def _(j):
            pltpu.sync_copy(rows_v, out_acc.at[pl.ds(s * ROWS_W + j * CH, CH)])

        for ch in (1 - c, c):
            baseA = s * E_C + ch * E_W
            pltpu.sync_copy(src_hbm.at[pl.ds(baseA, E_W)], src_v)
            pltpu.sync_copy(dst_hbm.at[pl.ds(baseA, E_W)], dst_v)

            @pl.loop(0, E_W // LANES)
            def _(i):
                sl = pl.ds(i * LANES, LANES)
                ss = src_v[sl]
                dd = dst_v[sl]
                e = plsc.load_gather(as_v, [ss]) + plsc.load_gather(ad_v, [dd])
                ex = jnp.exp(_leaky(e) - shift)
                exv[sl] = ex
                plsc.addupdate_scatter(denom_v, [dd], ex)

        pltpu.sync_copy(denom_v, denom_stage.at[s])
        plsc.subcore_barrier()
        col = s * ROWS_W
        pltpu.sync_copy(denom_stage.at[0, pl.ds(col, ROWS_W)], acc_v)
        for w in range(1, NS):
            pltpu.sync_copy(denom_stage.at[w, pl.ds(col, ROWS_W)], tmp_v)

            @pl.loop(0, ROWS_W // LANES)
            def _(i):
                sl = pl.ds(i * LANES, LANES)
                acc_v[sl] = acc_v[sl] + tmp_v[sl]

        pltpu.sync_copy(acc_v, denom_fin.at[pl.ds(col, ROWS_W)])
        plsc.subcore_barrier()
        pltpu.sync_copy(denom_fin, denom_v)

        baseB = s * E_C + c * E_W
        rowB = s * (E_C // CH) + c * (E_W // CH)
        pltpu.sync_copy(ew_hbm.at[pl.ds(baseB, E_W)], ew_v)
        pltpu.sync_copy(dst2_hbm.at[pl.ds(rowB, NCH)], dst2_v)

        @pl.loop(0, NCH)
        def _(j):
            eb = j * CH
            pltpu.sync_copy(h_hbm.at[src_v.at[pl.ds(eb, CH)]], rows_v)

            @pl.loop(0, CH // LANES)
            def _(i):
                sl = pl.ds(eb + i * LANES, LANES)
                dd = dst_v[sl]
                den = plsc.load_gather(denom_v, [dd])
                coeff_v[pl.ds(i * LANES, LANES)] = (
                    exv[sl] * ew_v[sl] / (den + 1e-16))

            @pl.loop(0, CH)
            def _(e2):
                spl = plsc.load_gather(coeff_v, [lax.broadcast(e2, (LANES,))])

                @pl.loop(0, D // LANES)
                def _(k2):
                    sl2 = pl.ds(k2 * LANES, LANES)
                    rows_v[e2, sl2] = rows_v[e2, sl2] * spl

            pltpu.sync_copy(rows_v, out_acc.at[dst2_v.at[j]], add=True)

        plsc.subcore_barrier()

        @pl.loop(0, ROWS_W // CH)
        def _(j):
            r0 = s * ROWS_W + j * CH
            pltpu.sync_copy(out_acc.at[pl.ds(r0, CH)],
                            out_hbm.at[c, pl.ds(r0, CH)])

    return sc_edge


_sc_edge_hid = _make_sc_edge(D_HID)
_sc_edge_out = _make_sc_edge(D_OUT)


def kernel(x, edge_index, edge_weight, W1, a1_src, a1_dst, b1,
           W2, a2_src, a2_dst, b2):
    src = edge_index[0].astype(jnp.int32)
    dst = edge_index[1].astype(jnp.int32)
    dst2 = dst.reshape(N_EDGES // CH, CH)
    ew = edge_weight.astype(jnp.float32)

    h1, av1, bv1, sh1 = _tc1(x, W1, a1_src, a1_dst)
    p1 = _sc_edge_hid(src, dst, dst2, ew, av1, bv1, sh1, h1)
    h2, av2, bv2, sh2 = _tc2(p1, b1, W2, a2_src, a2_dst)
    p2 = _sc_edge_out(src, dst, dst2, ew, av2, bv2, sh2, h2)
    return _tc3(p2, b2)


# speedup vs baseline: 20.6607x; 20.6607x over previous
import dataclasses
import functools

import jax
import jax.numpy as jnp
from jax import lax
from jax.experimental import pallas as pl
from jax.experimental.pallas import tpu as pltpu
from jax.experimental.pallas import tpu_sc as plsc

N_NODES = 10000
N_PAD = 10240
N_EDGES = 320000
D_IN = 128
D_HID = 128
D_OUT = 64

NC, NS, LANES = 2, 16, 16
E_T = N_EDGES // NS
CH = 80
NCH = E_T // CH
SUP = 2000
NSUP = E_T // SUP
RPS = SUP // CH
ROWS_W = N_PAD // NS


def _leaky(v):
    return jnp.where(v >= 0, v, 0.2 * v)


def _dense_tail(h, as_vec, ad_vec, d2, h_ref, av_ref, bv_ref, sh_ref):
    h_ref[0] = h[:, :d2]
    h_ref[1] = h[:, d2:]
    av = jnp.sum(h * as_vec[None, :], axis=1)
    bv = jnp.sum(h * ad_vec[None, :], axis=1)
    av_ref[...] = av
    bv_ref[...] = bv
    m = jnp.max(av) + jnp.max(bv)
    sh_ref[...] = jnp.full((LANES,), _leaky(m), jnp.float32)


def _tc1_body(x_ref, w_ref, as_ref, ad_ref, h_ref, av_ref, bv_ref, sh_ref):
    h = jnp.dot(x_ref[...], w_ref[...], preferred_element_type=jnp.float32)
    _dense_tail(h, as_ref[...], ad_ref[...], D_HID // 2,
                h_ref, av_ref, bv_ref, sh_ref)


def _tc2_body(p_ref, b_ref, w_ref, as_ref, ad_ref, h_ref, av_ref, bv_ref,
              sh_ref):
    xx = jnp.concatenate(
        [p_ref[0, :N_NODES, :], p_ref[1, :N_NODES, :]], axis=1)
    xx = jnp.maximum(xx + b_ref[...][None, :], 0.0)
    h = jnp.dot(xx, w_ref[...], preferred_element_type=jnp.float32)
    _dense_tail(h, as_ref[...], ad_ref[...], D_OUT // 2,
                h_ref, av_ref, bv_ref, sh_ref)


def _tc3_body(p_ref, b_ref, o_ref):
    o_ref[...] = jnp.concatenate(
        [p_ref[0, :N_NODES, :], p_ref[1, :N_NODES, :]], axis=1) \
        + b_ref[...][None, :]


def _dense_out(d):
    return [
        jax.ShapeDtypeStruct((NC, N_NODES, d // 2), jnp.float32),
        jax.ShapeDtypeStruct((N_NODES,), jnp.float32),
        jax.ShapeDtypeStruct((N_NODES,), jnp.float32),
        jax.ShapeDtypeStruct((LANES,), jnp.float32),
    ]


_tc1 = pl.pallas_call(_tc1_body, out_shape=_dense_out(D_HID))
_tc2 = pl.pallas_call(_tc2_body, out_shape=_dense_out(D_OUT))
_tc3 = pl.pallas_call(
    _tc3_body, out_shape=jax.ShapeDtypeStruct((N_NODES, D_OUT), jnp.float32))


def _sc_compiler_params():
    cp = pltpu.CompilerParams(use_tc_tiling_on_sc=False)
    if "needs_layout_passes" in pltpu.CompilerParams.__dataclass_fields__:
        cp = dataclasses.replace(cp, needs_layout_passes=False)
    return cp


def _make_sc_edge(D):
    D2 = D // 2
    mesh = plsc.VectorSubcoreMesh(core_axis_name="c", subcore_axis_name="s")

    @functools.partial(
        pl.kernel,
        compiler_params=_sc_compiler_params(),
        out_type=jax.ShapeDtypeStruct((NC, N_PAD, D2), jnp.float32),
        mesh=mesh,
        scratch_types=[
            pltpu.VMEM((N_PAD,), jnp.float32),
            pltpu.VMEM((N_PAD,), jnp.float32),
            pltpu.VMEM((N_PAD,), jnp.float32),
            pltpu.VMEM((SUP,), jnp.int32),
            pltpu.VMEM((SUP,), jnp.float32),
            pltpu.VMEM((NCH, CH), jnp.int32),
            pltpu.VMEM((CH, D // 2), jnp.float32),
            pltpu.VMEM((CH,), jnp.float32),
            pltpu.VMEM((ROWS_W,), jnp.float32),
            pltpu.VMEM((ROWS_W,), jnp.float32),
            pltpu.VMEM((LANES,), jnp.float32),
            pltpu.VMEM_SHARED((NS, N_PAD), jnp.float32),
            pltpu.VMEM_SHARED((N_PAD,), jnp.float32),
            pltpu.VMEM_SHARED((N_PAD, D2), jnp.float32),
        ],
    )
    def sc_edge(src_hbm, dst2_hbm, ew_hbm, as_hbm, ad_hbm, sh_hbm,
                h_hbm, out_hbm,
                as_v, ad_v, denom_v, src_s, ew_s, dst2_s, rows_v,
                coeff_v, acc_v, tmp_v, shift_v, denom_stage, denom_fin,
                out_acc):
        c = lax.axis_index("c")
        s = lax.axis_index("s")
        zero16 = jnp.zeros((LANES,), jnp.float32)

        pltpu.sync_copy(as_hbm, as_v.at[pl.ds(0, N_NODES)])
        pltpu.sync_copy(ad_hbm, ad_v.at[pl.ds(0, N_NODES)])
        pltpu.sync_copy(sh_hbm, shift_v)
        pltpu.sync_copy(dst2_hbm.at[s], dst2_s)
        shift = shift_v[...]

        @pl.loop(0, N_PAD // LANES)
        def _(i):
            denom_v[pl.ds(i * LANES, LANES)] = zero16

        @pl.loop(0, CH)
        def _(e2):
            @pl.loop(0, D2 // LANES)
            def _(k2):
                rows_v[e2, pl.ds(k2 * LANES, LANES)] = zero16

        @pl.loop(0, ROWS_W // CH)
        def _(j):
            r0 = pl.multiple_of(s * ROWS_W + j * CH, CH)
            pltpu.sync_copy(rows_v, out_acc.at[pl.ds(r0, CH)])

        @pl.loop(0, NSUP)
        def _(g):
            base = pl.multiple_of(s * E_T + g * SUP, SUP)
            pltpu.sync_copy(src_hbm.at[pl.ds(base, SUP)], src_s)

            @pl.loop(0, RPS)
            def _(r2):
                gr = g * RPS + r2
                for i in range(CH // LANES):
                    ss = src_s[pl.ds(r2 * CH + i * LANES, LANES)]
                    dd = dst2_s[gr, pl.ds(i * LANES, LANES)]
                    e = (plsc.load_gather(as_v, [ss])
                         + plsc.load_gather(ad_v, [dd]))
                    ex = jnp.exp(_leaky(e) - shift)
                    plsc.addupdate_scatter(denom_v, [dd], ex)

        pltpu.sync_copy(denom_v, denom_stage.at[s])
        plsc.subcore_barrier()
        col = pl.multiple_of(s * ROWS_W, ROWS_W)
        pltpu.sync_copy(denom_stage.at[0, pl.ds(col, ROWS_W)], acc_v)
        for w in range(1, NS):
            pltpu.sync_copy(denom_stage.at[w, pl.ds(col, ROWS_W)], tmp_v)

            @pl.loop(0, ROWS_W // LANES)
            def _(i):
                sl = pl.ds(i * LANES, LANES)
                acc_v[sl] = acc_v[sl] + tmp_v[sl]

        pltpu.sync_copy(acc_v, denom_fin.at[pl.ds(col, ROWS_W)])
        plsc.subcore_barrier()
        pltpu.sync_copy(denom_fin, denom_v)

        def pass_b(h_pl):
            @pl.loop(0, NSUP)
            def _(g):
                base = pl.multiple_of(s * E_T + g * SUP, SUP)
                pltpu.sync_copy(src_hbm.at[pl.ds(base, SUP)], src_s)
                pltpu.sync_copy(ew_hbm.at[pl.ds(base, SUP)], ew_s)

                @pl.loop(0, RPS)
                def _(r2):
                    gr = g * RPS + r2
                    pltpu.sync_copy(
                        h_pl.at[src_s.at[pl.ds(r2 * CH, CH)]], rows_v)
                    for i in range(CH // LANES):
                        sl = pl.ds(r2 * CH + i * LANES, LANES)
                        ss = src_s[sl]
                        dd = dst2_s[gr, pl.ds(i * LANES, LANES)]
                        e = (plsc.load_gather(as_v, [ss])
                             + plsc.load_gather(ad_v, [dd]))
                        ex = jnp.exp(_leaky(e) - shift)
                        den = plsc.load_gather(denom_v, [dd])
                        coeff_v[pl.ds(i * LANES, LANES)] = (
                            ex * ew_s[sl] / (den + 1e-16))

                    @pl.loop(0, CH)
                    def _(e2):
                        spl = plsc.load_gather(
                            coeff_v, [lax.broadcast(e2, (LANES,))])

                        @pl.loop(0, D2 // LANES)
                        def _(k2):
                            sl2 = pl.ds(k2 * LANES, LANES)
                            rows_v[e2, sl2] = rows_v[e2, sl2] * spl

                    pltpu.sync_copy(rows_v, out_acc.at[dst2_s.at[gr]],
                                    add=True)

        @pl.when(c == 0)
        def _():
            pass_b(h_hbm.at[0])

        @pl.when(c == 1)
        def _():
            pass_b(h_hbm.at[1])

        plsc.subcore_barrier()

        def copy_out(out_pl):
            @pl.loop(0, ROWS_W // CH)
            def _(j):
                r0 = pl.multiple_of(s * ROWS_W + j * CH, CH)
                pltpu.sync_copy(out_acc.at[pl.ds(r0, CH)],
                                out_pl.at[pl.ds(r0, CH)])

        @pl.when(c == 0)
        def _():
            copy_out(out_hbm.at[0])

        @pl.when(c == 1)
        def _():
            copy_out(out_hbm.at[1])

    return sc_edge


_sc_edge_hid = _make_sc_edge(D_HID)
_sc_edge_out = _make_sc_edge(D_OUT)


def kernel(x, edge_index, edge_weight, W1, a1_src, a1_dst, b1,
           W2, a2_src, a2_dst, b2):
    src = edge_index[0].astype(jnp.int32)
    dst = edge_index[1].astype(jnp.int32)
    dst2 = dst.reshape(NS, NCH, CH)
    ew = edge_weight.astype(jnp.float32)

    h1, av1, bv1, sh1 = _tc1(x, W1, a1_src, a1_dst)
    p1 = _sc_edge_hid(src, dst2, ew, av1, bv1, sh1, h1)
    h2, av2, bv2, sh2 = _tc2(p1, b1, W2, a2_src, a2_dst)
    p2 = _sc_edge_out(src, dst2, ew, av2, bv2, sh2, h2)
    return _tc3(p2, b2)


# speedup vs baseline: 31.9667x; 1.5472x over previous
import dataclasses
import functools

import jax
import jax.numpy as jnp
from jax import lax
from jax.experimental import pallas as pl
from jax.experimental.pallas import tpu as pltpu
from jax.experimental.pallas import tpu_sc as plsc

N_NODES = 10000
N_PAD = 10240
N_EDGES = 320000
D_IN = 128
D_HID = 128
D_OUT = 64

NC, NS, LANES = 2, 16, 16
E_T = N_EDGES // NS
CH = 80
NCH = E_T // CH
SUP = 4000
NSUP = E_T // SUP
RPS = SUP // CH
ROWS_W = N_PAD // NS


def _leaky(v):
    return jnp.where(v >= 0, v, 0.2 * v)


def _dense_tail(h, as_vec, ad_vec, d2, h_ref, av_ref, bv_ref, sh_ref):
    h_ref[0] = h[:, :d2]
    h_ref[1] = h[:, d2:]
    av = jnp.sum(h * as_vec[None, :], axis=1)
    bv = jnp.sum(h * ad_vec[None, :], axis=1)
    av_ref[...] = av
    bv_ref[...] = bv
    m = jnp.max(av) + jnp.max(bv)
    sh_ref[...] = jnp.full((LANES,), _leaky(m), jnp.float32)


def _tc1_body(x_ref, w_ref, as_ref, ad_ref, h_ref, av_ref, bv_ref, sh_ref):
    h = jnp.dot(x_ref[...], w_ref[...], preferred_element_type=jnp.float32)
    _dense_tail(h, as_ref[...], ad_ref[...], D_HID // 2,
                h_ref, av_ref, bv_ref, sh_ref)


def _tc2_body(p_ref, b_ref, w_ref, as_ref, ad_ref, h_ref, av_ref, bv_ref,
              sh_ref):
    xx = jnp.concatenate(
        [p_ref[0, :N_NODES, :], p_ref[1, :N_NODES, :]], axis=1)
    xx = jnp.maximum(xx + b_ref[...][None, :], 0.0)
    h = jnp.dot(xx, w_ref[...], preferred_element_type=jnp.float32)
    _dense_tail(h, as_ref[...], ad_ref[...], D_OUT // 2,
                h_ref, av_ref, bv_ref, sh_ref)


def _tc3_body(p_ref, b_ref, o_ref):
    o_ref[...] = jnp.concatenate(
        [p_ref[0, :N_NODES, :], p_ref[1, :N_NODES, :]], axis=1) \
        + b_ref[...][None, :]


def _dense_out(d):
    return [
        jax.ShapeDtypeStruct((NC, N_NODES, d // 2), jnp.float32),
        jax.ShapeDtypeStruct((N_NODES,), jnp.float32),
        jax.ShapeDtypeStruct((N_NODES,), jnp.float32),
        jax.ShapeDtypeStruct((LANES,), jnp.float32),
    ]


_tc1 = pl.pallas_call(_tc1_body, out_shape=_dense_out(D_HID))
_tc2 = pl.pallas_call(_tc2_body, out_shape=_dense_out(D_OUT))
_tc3 = pl.pallas_call(
    _tc3_body, out_shape=jax.ShapeDtypeStruct((N_NODES, D_OUT), jnp.float32))


def _sc_compiler_params():
    cp = pltpu.CompilerParams(use_tc_tiling_on_sc=False)
    if "needs_layout_passes" in pltpu.CompilerParams.__dataclass_fields__:
        cp = dataclasses.replace(cp, needs_layout_passes=False)
    return cp


def _make_sc_edge(D):
    D2 = D // 2
    mesh = plsc.VectorSubcoreMesh(core_axis_name="c", subcore_axis_name="s")

    @functools.partial(
        pl.kernel,
        compiler_params=_sc_compiler_params(),
        out_type=jax.ShapeDtypeStruct((NC, N_PAD, D2), jnp.float32),
        mesh=mesh,
        scratch_types=[
            pltpu.VMEM((N_PAD,), jnp.float32),
            pltpu.VMEM((N_PAD,), jnp.float32),
            pltpu.VMEM((N_PAD,), jnp.float32),
            pltpu.VMEM((SUP,), jnp.int32),
            pltpu.VMEM((SUP,), jnp.float32),
            pltpu.VMEM((NCH, CH), jnp.int32),
            pltpu.VMEM((CH, D // 2), jnp.float32),
            pltpu.VMEM((CH, D // 2), jnp.float32),
            pltpu.VMEM((CH,), jnp.float32),
            pltpu.VMEM((ROWS_W,), jnp.float32),
            pltpu.VMEM((ROWS_W,), jnp.float32),
            pltpu.VMEM((LANES,), jnp.float32),
            pltpu.VMEM_SHARED((NS, N_PAD), jnp.float32),
            pltpu.VMEM_SHARED((N_PAD,), jnp.float32),
            pltpu.VMEM_SHARED((N_PAD, D2), jnp.float32),
            pltpu.SemaphoreType.DMA,
            pltpu.SemaphoreType.DMA,
            pltpu.SemaphoreType.DMA,
            pltpu.SemaphoreType.DMA,
        ],
    )
    def sc_edge(src_hbm, dst2_hbm, ew_hbm, as_hbm, ad_hbm, sh_hbm,
                h_hbm, out_hbm,
                as_v, ad_v, denom_v, src_s, ew_s, dst2_s, rows_a, rows_b,
                coeff_v, acc_v, tmp_v, shift_v, denom_stage, denom_fin,
                out_acc, gsem_a, gsem_b, ssem_a, ssem_b):
        c = lax.axis_index("c")
        s = lax.axis_index("s")
        zero16 = jnp.zeros((LANES,), jnp.float32)

        pltpu.sync_copy(as_hbm, as_v.at[pl.ds(0, N_NODES)])
        pltpu.sync_copy(ad_hbm, ad_v.at[pl.ds(0, N_NODES)])
        pltpu.sync_copy(sh_hbm, shift_v)
        pltpu.sync_copy(dst2_hbm.at[s], dst2_s)
        shift = shift_v[...]

        @pl.loop(0, N_PAD // LANES)
        def _(i):
            denom_v[pl.ds(i * LANES, LANES)] = zero16

        @pl.loop(0, CH)
        def _(e2):
            @pl.loop(0, D2 // LANES)
            def _(k2):
                rows_a[e2, pl.ds(k2 * LANES, LANES)] = zero16

        @pl.loop(0, ROWS_W // CH)
        def _(j):
            r0 = pl.multiple_of(s * ROWS_W + j * CH, CH)
            pltpu.sync_copy(rows_a, out_acc.at[pl.ds(r0, CH)])

        @pl.loop(0, NSUP)
        def _(g):
            base = pl.multiple_of(s * E_T + g * SUP, SUP)
            pltpu.sync_copy(src_hbm.at[pl.ds(base, SUP)], src_s)

            @pl.loop(0, RPS)
            def _(r2):
                gr = g * RPS + r2
                for i in range(CH // LANES):
                    ss = src_s[pl.ds(r2 * CH + i * LANES, LANES)]
                    dd = dst2_s[gr, pl.ds(i * LANES, LANES)]
                    e = (plsc.load_gather(as_v, [ss])
                         + plsc.load_gather(ad_v, [dd]))
                    ex = jnp.exp(_leaky(e) - shift)
                    plsc.addupdate_scatter(denom_v, [dd], ex)

        pltpu.sync_copy(denom_v, denom_stage.at[s])
        plsc.subcore_barrier()
        col = pl.multiple_of(s * ROWS_W, ROWS_W)
        pltpu.sync_copy(denom_stage.at[0, pl.ds(col, ROWS_W)], acc_v)
        for w in range(1, NS):
            pltpu.sync_copy(denom_stage.at[w, pl.ds(col, ROWS_W)], tmp_v)

            @pl.loop(0, ROWS_W // LANES)
            def _(i):
                sl = pl.ds(i * LANES, LANES)
                acc_v[sl] = acc_v[sl] + tmp_v[sl]

        pltpu.sync_copy(acc_v, denom_fin.at[pl.ds(col, ROWS_W)])
        plsc.subcore_barrier()
        pltpu.sync_copy(denom_fin, denom_v)

        def pass_b(h_pl):
            def g_start(r2, buf, sem):
                pltpu.async_copy(
                    h_pl.at[src_s.at[pl.ds(r2 * CH, CH)]], buf, sem)

            def g_wait(r2, buf, sem):
                pltpu.make_async_copy(
                    h_pl.at[src_s.at[pl.ds(r2 * CH, CH)]], buf, sem).wait()

            def s_start(gr, buf, sem):
                pltpu.async_copy(buf, out_acc.at[dst2_s.at[gr]], sem,
                                 add=True)

            def s_wait(gr, buf, sem):
                pltpu.make_async_copy(
                    buf, out_acc.at[dst2_s.at[gr]], sem).wait()

            def compute(r2, gr, buf):
                for i in range(CH // LANES):
                    sl = pl.ds(r2 * CH + i * LANES, LANES)
                    ss = src_s[sl]
                    dd = dst2_s[gr, pl.ds(i * LANES, LANES)]
                    e = (plsc.load_gather(as_v, [ss])
                         + plsc.load_gather(ad_v, [dd]))
                    ex = jnp.exp(_leaky(e) - shift)
                    den = plsc.load_gather(denom_v, [dd])
                    coeff_v[pl.ds(i * LANES, LANES)] = (
                        ex * ew_s[sl] / (den + 1e-16))

                @pl.loop(0, CH)
                def _(e2):
                    spl = plsc.load_gather(
                        coeff_v, [lax.broadcast(e2, (LANES,))])

                    @pl.loop(0, D2 // LANES)
                    def _(k2):
                        sl2 = pl.ds(k2 * LANES, LANES)
                        buf[e2, sl2] = buf[e2, sl2] * spl

            @pl.loop(0, NSUP)
            def _(g):
                base = pl.multiple_of(s * E_T + g * SUP, SUP)
                pltpu.sync_copy(src_hbm.at[pl.ds(base, SUP)], src_s)
                pltpu.sync_copy(ew_hbm.at[pl.ds(base, SUP)], ew_s)
                g_start(0, rows_a, gsem_a)

                @pl.loop(0, RPS // 2)
                def _(t):
                    ra = 2 * t
                    rb = 2 * t + 1
                    gra = g * RPS + ra
                    grb = g * RPS + rb
                    g_wait(ra, rows_a, gsem_a)

                    @pl.when(t > 0)
                    def _():
                        s_wait(grb - 2, rows_b, ssem_b)

                    g_start(rb, rows_b, gsem_b)
                    compute(ra, gra, rows_a)
                    s_start(gra, rows_a, ssem_a)
                    g_wait(rb, rows_b, gsem_b)
                    s_wait(gra, rows_a, ssem_a)

                    @pl.when(t < RPS // 2 - 1)
                    def _():
                        g_start(ra + 2, rows_a, gsem_a)

                    compute(rb, grb, rows_b)
                    s_start(grb, rows_b, ssem_b)

                s_wait(RPS - 1 + g * RPS, rows_b, ssem_b)

        @pl.when(c == 0)
        def _():
            pass_b(h_hbm.at[0])

        @pl.when(c == 1)
        def _():
            pass_b(h_hbm.at[1])

        plsc.subcore_barrier()

        def copy_out(out_pl):
            @pl.loop(0, ROWS_W // CH)
            def _(j):
                r0 = pl.multiple_of(s * ROWS_W + j * CH, CH)
                pltpu.sync_copy(out_acc.at[pl.ds(r0, CH)],
                                out_pl.at[pl.ds(r0, CH)])

        @pl.when(c == 0)
        def _():
            copy_out(out_hbm.at[0])

        @pl.when(c == 1)
        def _():
            copy_out(out_hbm.at[1])

    return sc_edge


_sc_edge_hid = _make_sc_edge(D_HID)
_sc_edge_out = _make_sc_edge(D_OUT)


def kernel(x, edge_index, edge_weight, W1, a1_src, a1_dst, b1,
           W2, a2_src, a2_dst, b2):
    src = edge_index[0].astype(jnp.int32)
    dst = edge_index[1].astype(jnp.int32)
    dst2 = dst.reshape(NS, NCH, CH)
    ew = edge_weight.astype(jnp.float32)

    h1, av1, bv1, sh1 = _tc1(x, W1, a1_src, a1_dst)
    p1 = _sc_edge_hid(src, dst2, ew, av1, bv1, sh1, h1)
    h2, av2, bv2, sh2 = _tc2(p1, b1, W2, a2_src, a2_dst)
    p2 = _sc_edge_out(src, dst2, ew, av2, bv2, sh2, h2)
    return _tc3(p2, b2)


# speedup vs baseline: 35.9675x; 1.1252x over previous
import dataclasses
import functools

import jax
import jax.numpy as jnp
from jax import lax
from jax.experimental import pallas as pl
from jax.experimental.pallas import tpu as pltpu
from jax.experimental.pallas import tpu_sc as plsc

N_NODES = 10000
N_PAD = 10240
N_EDGES = 320000
D_IN = 128
D_HID = 128
D_OUT = 64

NC, NS, LANES = 2, 16, 16
E_T = N_EDGES // NS
CH = 80
NCH = E_T // CH
SUP = 4000
NSUP = E_T // SUP
RPS = SUP // CH
ROWS_W = N_PAD // NS


def _leaky(v):
    return jnp.where(v >= 0, v, 0.2 * v)


def _dense_tail(h, as_vec, ad_vec, d2, h_ref, av_ref, bv_ref, sh_ref):
    h_ref[0] = h[:, :d2]
    h_ref[1] = h[:, d2:]
    av = jnp.sum(h * as_vec[None, :], axis=1)
    bv = jnp.sum(h * ad_vec[None, :], axis=1)
    av_ref[...] = av
    bv_ref[...] = bv
    m = jnp.max(av) + jnp.max(bv)
    sh_ref[...] = jnp.full((LANES,), _leaky(m), jnp.float32)


def _tc1_body(x_ref, w_ref, as_ref, ad_ref, h_ref, av_ref, bv_ref, sh_ref):
    h = jnp.dot(x_ref[...], w_ref[...], preferred_element_type=jnp.float32)
    _dense_tail(h, as_ref[...], ad_ref[...], D_HID // 2,
                h_ref, av_ref, bv_ref, sh_ref)


def _tc2_body(p_ref, b_ref, w_ref, as_ref, ad_ref, h_ref, av_ref, bv_ref,
              sh_ref):
    xx = jnp.concatenate(
        [p_ref[0, :N_NODES, :], p_ref[1, :N_NODES, :]], axis=1)
    xx = jnp.maximum(xx + b_ref[...][None, :], 0.0)
    h = jnp.dot(xx, w_ref[...], preferred_element_type=jnp.float32)
    _dense_tail(h, as_ref[...], ad_ref[...], D_OUT // 2,
                h_ref, av_ref, bv_ref, sh_ref)


def _tc3_body(p_ref, b_ref, o_ref):
    o_ref[...] = jnp.concatenate(
        [p_ref[0, :N_NODES, :], p_ref[1, :N_NODES, :]], axis=1) \
        + b_ref[...][None, :]


def _dense_out(d):
    return [
        jax.ShapeDtypeStruct((NC, N_NODES, d // 2), jnp.float32),
        jax.ShapeDtypeStruct((N_NODES,), jnp.float32),
        jax.ShapeDtypeStruct((N_NODES,), jnp.float32),
        jax.ShapeDtypeStruct((LANES,), jnp.float32),
    ]


_tc1 = pl.pallas_call(_tc1_body, out_shape=_dense_out(D_HID))
_tc2 = pl.pallas_call(_tc2_body, out_shape=_dense_out(D_OUT))
_tc3 = pl.pallas_call(
    _tc3_body, out_shape=jax.ShapeDtypeStruct((N_NODES, D_OUT), jnp.float32))


def _sc_compiler_params():
    cp = pltpu.CompilerParams(use_tc_tiling_on_sc=False)
    if "needs_layout_passes" in pltpu.CompilerParams.__dataclass_fields__:
        cp = dataclasses.replace(cp, needs_layout_passes=False)
    return cp


def _make_sc_edge(D):
    D2 = D // 2
    mesh = plsc.VectorSubcoreMesh(core_axis_name="c", subcore_axis_name="s")

    @functools.partial(
        pl.kernel,
        compiler_params=_sc_compiler_params(),
        out_type=jax.ShapeDtypeStruct((NC, N_PAD, D2), jnp.float32),
        mesh=mesh,
        scratch_types=[
            pltpu.VMEM((N_PAD,), jnp.float32),
            pltpu.VMEM((N_PAD,), jnp.float32),
            pltpu.VMEM((N_PAD,), jnp.float32),
            pltpu.VMEM((SUP,), jnp.int32),
            pltpu.VMEM((SUP,), jnp.float32),
            pltpu.VMEM((NCH, CH), jnp.int32),
            pltpu.VMEM((CH, D // 2), jnp.float32),
            pltpu.VMEM((CH, D // 2), jnp.float32),
            pltpu.VMEM((CH,), jnp.float32),
            pltpu.VMEM((ROWS_W,), jnp.float32),
            pltpu.VMEM((ROWS_W,), jnp.float32),
            pltpu.VMEM((LANES,), jnp.float32),
            pltpu.VMEM_SHARED((NS, N_PAD), jnp.float32),
            pltpu.VMEM_SHARED((N_PAD,), jnp.float32),
            pltpu.VMEM_SHARED((N_PAD, D2), jnp.float32),
            pltpu.SemaphoreType.DMA,
            pltpu.SemaphoreType.DMA,
            pltpu.SemaphoreType.DMA,
            pltpu.SemaphoreType.DMA,
        ],
    )
    def sc_edge(src_hbm, dst2_hbm, ew_hbm, as_hbm, ad_hbm, sh_hbm,
                h_hbm, out_hbm,
                as_v, ad_v, denom_v, src_s, ew_s, dst2_s, rows_a, rows_b,
                coeff_v, acc_v, tmp_v, shift_v, denom_stage, denom_fin,
                out_acc, gsem_a, gsem_b, ssem_a, ssem_b):
        c = lax.axis_index("c")
        s = lax.axis_index("s")
        zero16 = jnp.zeros((LANES,), jnp.float32)

        pltpu.sync_copy(as_hbm, as_v.at[pl.ds(0, N_NODES)])
        pltpu.sync_copy(ad_hbm, ad_v.at[pl.ds(0, N_NODES)])
        pltpu.sync_copy(sh_hbm, shift_v)
        pltpu.sync_copy(dst2_hbm.at[s], dst2_s)
        shift = shift_v[...]

        @pl.loop(0, N_PAD // LANES)
        def _(i):
            denom_v[pl.ds(i * LANES, LANES)] = zero16

        @pl.loop(0, CH)
        def _(e2):
            @pl.loop(0, D2 // LANES)
            def _(k2):
                rows_a[e2, pl.ds(k2 * LANES, LANES)] = zero16

        @pl.loop(0, ROWS_W // CH)
        def _(j):
            r0 = pl.multiple_of(s * ROWS_W + j * CH, CH)
            pltpu.sync_copy(rows_a, out_acc.at[pl.ds(r0, CH)])

        @pl.loop(0, NSUP)
        def _(g):
            base = pl.multiple_of(s * E_T + g * SUP, SUP)
            pltpu.sync_copy(src_hbm.at[pl.ds(base, SUP)], src_s)

            @pl.loop(0, RPS)
            def _(r2):
                gr = g * RPS + r2
                for i in range(CH // LANES):
                    ss = src_s[pl.ds(r2 * CH + i * LANES, LANES)]
                    dd = dst2_s[gr, pl.ds(i * LANES, LANES)]
                    e = (plsc.load_gather(as_v, [ss])
                         + plsc.load_gather(ad_v, [dd]))
                    ex = jnp.exp(_leaky(e) - shift)
                    plsc.addupdate_scatter(denom_v, [dd], ex)

        pltpu.sync_copy(denom_v, denom_stage.at[s])
        plsc.subcore_barrier()
        col = pl.multiple_of(s * ROWS_W, ROWS_W)
        pltpu.sync_copy(denom_stage.at[0, pl.ds(col, ROWS_W)], acc_v)
        for w in range(1, NS):
            pltpu.sync_copy(denom_stage.at[w, pl.ds(col, ROWS_W)], tmp_v)

            @pl.loop(0, ROWS_W // LANES)
            def _(i):
                sl = pl.ds(i * LANES, LANES)
                acc_v[sl] = acc_v[sl] + tmp_v[sl]

        pltpu.sync_copy(acc_v, denom_fin.at[pl.ds(col, ROWS_W)])
        plsc.subcore_barrier()
        pltpu.sync_copy(denom_fin, denom_v)

        def pass_b(h_pl):
            def g_start(r2, buf, sem):
                pltpu.async_copy(
                    h_pl.at[src_s.at[pl.ds(r2 * CH, CH)]], buf, sem)

            def g_wait(r2, buf, sem):
                pltpu.make_async_copy(
                    h_pl.at[src_s.at[pl.ds(r2 * CH, CH)]], buf, sem).wait()

            def s_start(gr, buf, sem):
                pltpu.async_copy(buf, out_acc.at[dst2_s.at[gr]], sem,
                                 add=True)

            def s_wait(gr, buf, sem):
                pltpu.make_async_copy(
                    buf, out_acc.at[dst2_s.at[gr]], sem).wait()

            def compute(r2, gr, buf):
                for i in range(CH // LANES):
                    sl = pl.ds(r2 * CH + i * LANES, LANES)
                    ss = src_s[sl]
                    dd = dst2_s[gr, pl.ds(i * LANES, LANES)]
                    e = (plsc.load_gather(as_v, [ss])
                         + plsc.load_gather(ad_v, [dd]))
                    ex = jnp.exp(_leaky(e) - shift)
                    den = plsc.load_gather(denom_v, [dd])
                    coeff_v[pl.ds(i * LANES, LANES)] = (
                        ex * ew_s[sl] / (den + 1e-16))

                @plsc.parallel_loop(0, CH, unroll=4)
                def _(e2):
                    spl = plsc.load_gather(
                        coeff_v, [lax.broadcast(e2, (LANES,))])
                    for k2 in range(D2 // LANES):
                        sl2 = pl.ds(k2 * LANES, LANES)
                        buf[e2, sl2] = buf[e2, sl2] * spl

            @pl.loop(0, NSUP)
            def _(g):
                base = pl.multiple_of(s * E_T + g * SUP, SUP)
                pltpu.sync_copy(src_hbm.at[pl.ds(base, SUP)], src_s)
                pltpu.sync_copy(ew_hbm.at[pl.ds(base, SUP)], ew_s)
                g_start(0, rows_a, gsem_a)

                @pl.loop(0, RPS // 2)
                def _(t):
                    ra = 2 * t
                    rb = 2 * t + 1
                    gra = g * RPS + ra
                    grb = g * RPS + rb
                    g_wait(ra, rows_a, gsem_a)

                    @pl.when(t > 0)
                    def _():
                        s_wait(grb - 2, rows_b, ssem_b)

                    g_start(rb, rows_b, gsem_b)
                    compute(ra, gra, rows_a)
                    s_start(gra, rows_a, ssem_a)
                    g_wait(rb, rows_b, gsem_b)
                    s_wait(gra, rows_a, ssem_a)

                    @pl.when(t < RPS // 2 - 1)
                    def _():
                        g_start(ra + 2, rows_a, gsem_a)

                    compute(rb, grb, rows_b)
                    s_start(grb, rows_b, ssem_b)

                s_wait(RPS - 1 + g * RPS, rows_b, ssem_b)

        @pl.when(c == 0)
        def _():
            pass_b(h_hbm.at[0])

        @pl.when(c == 1)
        def _():
            pass_b(h_hbm.at[1])

        plsc.subcore_barrier()

        def copy_out(out_pl):
            @pl.loop(0, ROWS_W // CH)
            def _(j):
                r0 = pl.multiple_of(s * ROWS_W + j * CH, CH)
                pltpu.sync_copy(out_acc.at[pl.ds(r0, CH)],
                                out_pl.at[pl.ds(r0, CH)])

        @pl.when(c == 0)
        def _():
            copy_out(out_hbm.at[0])

        @pl.when(c == 1)
        def _():
            copy_out(out_hbm.at[1])

    return sc_edge


_sc_edge_hid = _make_sc_edge(D_HID)
_sc_edge_out = _make_sc_edge(D_OUT)


def kernel(x, edge_index, edge_weight, W1, a1_src, a1_dst, b1,
           W2, a2_src, a2_dst, b2):
    src = edge_index[0].astype(jnp.int32)
    dst = edge_index[1].astype(jnp.int32)
    dst2 = dst.reshape(NS, NCH, CH)
    ew = edge_weight.astype(jnp.float32)

    h1, av1, bv1, sh1 = _tc1(x, W1, a1_src, a1_dst)
    p1 = _sc_edge_hid(src, dst2, ew, av1, bv1, sh1, h1)
    h2, av2, bv2, sh2 = _tc2(p1, b1, W2, a2_src, a2_dst)
    p2 = _sc_edge_out(src, dst2, ew, av2, bv2, sh2, h2)
    return _tc3(p2, b2)


# speedup vs baseline: 36.3490x; 1.0106x over previous
import dataclasses
import functools

import jax
import jax.numpy as jnp
from jax import lax
from jax.experimental import pallas as pl
from jax.experimental.pallas import tpu as pltpu
from jax.experimental.pallas import tpu_sc as plsc

N_NODES = 10000
N_PAD = 10240
N_EDGES = 320000
D_IN = 128
D_HID = 128
D_OUT = 64

NC, NS, LANES = 2, 16, 16
E_T = N_EDGES // NS
CH = 80
NCH = E_T // CH
SUP = 4000
NSUP = E_T // SUP
RPS = SUP // CH
ROWS_W = N_PAD // NS


def _leaky(v):
    return jnp.where(v >= 0, v, 0.2 * v)


def _dense_tail(h, as_vec, ad_vec, d2, h_ref, av_ref, bv_ref, sh_ref):
    h_ref[0] = h[:, :d2]
    h_ref[1] = h[:, d2:]
    av = jnp.sum(h * as_vec[None, :], axis=1)
    bv = jnp.sum(h * ad_vec[None, :], axis=1)
    av_ref[...] = av
    bv_ref[...] = bv
    m = jnp.max(av) + jnp.max(bv)
    sh_ref[...] = jnp.full((LANES,), _leaky(m), jnp.float32)


def _tc1_body(x_ref, w_ref, as_ref, ad_ref, h_ref, av_ref, bv_ref, sh_ref):
    h = jnp.dot(x_ref[...], w_ref[...], preferred_element_type=jnp.float32)
    _dense_tail(h, as_ref[...], ad_ref[...], D_HID // 2,
                h_ref, av_ref, bv_ref, sh_ref)


def _tc2_body(p_ref, b_ref, w_ref, as_ref, ad_ref, h_ref, av_ref, bv_ref,
              sh_ref):
    xx = jnp.concatenate(
        [p_ref[0, :N_NODES, :], p_ref[1, :N_NODES, :]], axis=1)
    xx = jnp.maximum(xx + b_ref[...][None, :], 0.0)
    h = jnp.dot(xx, w_ref[...], preferred_element_type=jnp.float32)
    _dense_tail(h, as_ref[...], ad_ref[...], D_OUT // 2,
                h_ref, av_ref, bv_ref, sh_ref)


def _tc3_body(p_ref, b_ref, o_ref):
    o_ref[...] = jnp.concatenate(
        [p_ref[0, :N_NODES, :], p_ref[1, :N_NODES, :]], axis=1) \
        + b_ref[...][None, :]


def _dense_out(d):
    return [
        jax.ShapeDtypeStruct((NC, N_NODES, d // 2), jnp.float32),
        jax.ShapeDtypeStruct((N_NODES,), jnp.float32),
        jax.ShapeDtypeStruct((N_NODES,), jnp.float32),
        jax.ShapeDtypeStruct((LANES,), jnp.float32),
    ]


_tc1 = pl.pallas_call(_tc1_body, out_shape=_dense_out(D_HID))
_tc2 = pl.pallas_call(_tc2_body, out_shape=_dense_out(D_OUT))
_tc3 = pl.pallas_call(
    _tc3_body, out_shape=jax.ShapeDtypeStruct((N_NODES, D_OUT), jnp.float32))


def _sc_compiler_params():
    cp = pltpu.CompilerParams(use_tc_tiling_on_sc=False)
    if "needs_layout_passes" in pltpu.CompilerParams.__dataclass_fields__:
        cp = dataclasses.replace(cp, needs_layout_passes=False)
    return cp


def _make_sc_edge(D):
    D2 = D // 2
    mesh = plsc.VectorSubcoreMesh(core_axis_name="c", subcore_axis_name="s")

    @functools.partial(
        pl.kernel,
        compiler_params=_sc_compiler_params(),
        out_type=jax.ShapeDtypeStruct((NC, N_PAD, D2), jnp.float32),
        mesh=mesh,
        scratch_types=[
            pltpu.VMEM((N_PAD,), jnp.float32),
            pltpu.VMEM((N_PAD,), jnp.float32),
            pltpu.VMEM((N_PAD,), jnp.float32),
            pltpu.VMEM((SUP,), jnp.int32),
            pltpu.VMEM((SUP,), jnp.float32),
            pltpu.VMEM((NS, ROWS_W // 2), jnp.float32),
            pltpu.VMEM((NCH, CH), jnp.int32),
            pltpu.VMEM((CH, D // 2), jnp.float32),
            pltpu.VMEM((CH, D // 2), jnp.float32),
            pltpu.VMEM((CH,), jnp.float32),
            pltpu.VMEM((ROWS_W,), jnp.float32),
            pltpu.VMEM((LANES,), jnp.float32),
            pltpu.VMEM_SHARED((NS, N_PAD), jnp.float32),
            pltpu.VMEM_SHARED((N_PAD,), jnp.float32),
            pltpu.VMEM_SHARED((N_PAD, D2), jnp.float32),
            pltpu.SemaphoreType.DMA,
            pltpu.SemaphoreType.DMA,
            pltpu.SemaphoreType.DMA,
            pltpu.SemaphoreType.DMA,
            pltpu.SemaphoreType.DMA,
        ],
    )
    def sc_edge(src_hbm, dst2_hbm, ew_hbm, as_hbm, ad_hbm, sh_hbm,
                h_hbm, out_hbm,
                as_v, ad_v, denom_v, src_s, ew_s, red_v,
                dst2_s, rows_a, rows_b,
                coeff_v, acc_v, shift_v, denom_stage, denom_fin,
                out_acc, gsem_a, gsem_b, ssem_a, ssem_b, stsem):
        c = lax.axis_index("c")
        s = lax.axis_index("s")
        zero16 = jnp.zeros((LANES,), jnp.float32)

        pltpu.sync_copy(as_hbm, as_v.at[pl.ds(0, N_NODES)])
        pltpu.sync_copy(ad_hbm, ad_v.at[pl.ds(0, N_NODES)])
        pltpu.sync_copy(sh_hbm, shift_v)
        pltpu.sync_copy(dst2_hbm.at[s], dst2_s)
        shift = shift_v[...]

        @pl.loop(0, N_PAD // LANES)
        def _(i):
            denom_v[pl.ds(i * LANES, LANES)] = zero16

        @pl.loop(0, CH)
        def _(e2):
            @pl.loop(0, D2 // LANES)
            def _(k2):
                rows_a[e2, pl.ds(k2 * LANES, LANES)] = zero16

        @pl.loop(0, ROWS_W // CH)
        def _(j):
            r0 = pl.multiple_of(s * ROWS_W + j * CH, CH)
            pltpu.sync_copy(rows_a, out_acc.at[pl.ds(r0, CH)])

        @pl.loop(0, NSUP)
        def _(g):
            base = pl.multiple_of(s * E_T + g * SUP, SUP)
            pltpu.sync_copy(src_hbm.at[pl.ds(base, SUP)], src_s)

            @pl.loop(0, RPS)
            def _(r2):
                gr = g * RPS + r2
                for i in range(CH // LANES):
                    ss = src_s[pl.ds(r2 * CH + i * LANES, LANES)]
                    dd = dst2_s[gr, pl.ds(i * LANES, LANES)]
                    e = (plsc.load_gather(as_v, [ss])
                         + plsc.load_gather(ad_v, [dd]))
                    ex = jnp.exp(_leaky(e) - shift)
                    plsc.addupdate_scatter(denom_v, [dd], ex)

        pltpu.sync_copy(denom_v, denom_stage.at[s])
        plsc.subcore_barrier()
        col = pl.multiple_of(s * ROWS_W, ROWS_W)
        HW = ROWS_W // 2
        for half in range(2):
            colh = pl.multiple_of(col + half * HW, HW)
            for w in range(NS):
                pltpu.async_copy(denom_stage.at[w, pl.ds(colh, HW)],
                                 red_v.at[w], stsem)
            for w in range(NS):
                pltpu.make_async_copy(denom_stage.at[w, pl.ds(colh, HW)],
                                      red_v.at[w], stsem).wait()

            @plsc.parallel_loop(0, HW // LANES, unroll=2)
            def _(i):
                sl = pl.ds(i * LANES, LANES)
                t0 = red_v[0, sl] + red_v[1, sl]
                t1 = red_v[2, sl] + red_v[3, sl]
                t2 = red_v[4, sl] + red_v[5, sl]
                t3 = red_v[6, sl] + red_v[7, sl]
                t4 = red_v[8, sl] + red_v[9, sl]
                t5 = red_v[10, sl] + red_v[11, sl]
                t6 = red_v[12, sl] + red_v[13, sl]
                t7 = red_v[14, sl] + red_v[15, sl]
                acc_v[pl.ds(half * HW + i * LANES, LANES)] = (
                    ((t0 + t1) + (t2 + t3)) + ((t4 + t5) + (t6 + t7)))

        pltpu.sync_copy(acc_v, denom_fin.at[pl.ds(col, ROWS_W)])
        plsc.subcore_barrier()
        pltpu.sync_copy(denom_fin, denom_v)

        def pass_b(h_pl):
            def g_start(sv, r2, buf, sem):
                pltpu.async_copy(
                    h_pl.at[sv.at[pl.ds(r2 * CH, CH)]], buf, sem)

            def g_wait(sv, r2, buf, sem):
                pltpu.make_async_copy(
                    h_pl.at[sv.at[pl.ds(r2 * CH, CH)]], buf, sem).wait()

            def s_start(gr, buf, sem):
                pltpu.async_copy(buf, out_acc.at[dst2_s.at[gr]], sem,
                                 add=True)

            def s_wait(gr, buf, sem):
                pltpu.make_async_copy(
                    buf, out_acc.at[dst2_s.at[gr]], sem).wait()

            def compute(sv, ev, r2, gr, buf):
                for i in range(CH // LANES):
                    sl = pl.ds(r2 * CH + i * LANES, LANES)
                    ss = sv[sl]
                    dd = dst2_s[gr, pl.ds(i * LANES, LANES)]
                    e = (plsc.load_gather(as_v, [ss])
                         + plsc.load_gather(ad_v, [dd]))
                    ex = jnp.exp(_leaky(e) - shift)
                    den = plsc.load_gather(denom_v, [dd])
                    coeff_v[pl.ds(i * LANES, LANES)] = (
                        ex * ev[sl] / (den + 1e-16))

                @plsc.parallel_loop(0, CH, unroll=4)
                def _(e2):
                    spl = plsc.load_gather(
                        coeff_v, [lax.broadcast(e2, (LANES,))])
                    for k2 in range(D2 // LANES):
                        sl2 = pl.ds(k2 * LANES, LANES)
                        buf[e2, sl2] = buf[e2, sl2] * spl

            def stage_start(g, sv, ev):
                base = pl.multiple_of(s * E_T + g * SUP, SUP)
                pltpu.async_copy(src_hbm.at[pl.ds(base, SUP)], sv, stsem)
                pltpu.async_copy(ew_hbm.at[pl.ds(base, SUP)], ev, stsem)

            def stage_wait(g, sv, ev):
                base = pl.multiple_of(s * E_T + g * SUP, SUP)
                pltpu.make_async_copy(
                    src_hbm.at[pl.ds(base, SUP)], sv, stsem).wait()
                pltpu.make_async_copy(
                    ew_hbm.at[pl.ds(base, SUP)], ev, stsem).wait()

            stage_start(0, src_s, ew_s)
            for g in range(NSUP):
                sv, ev = src_s, ew_s
                if g > 0:
                    stage_start(g, sv, ev)
                stage_wait(g, sv, ev)
                g_start(sv, 0, rows_a, gsem_a)

                @pl.loop(0, RPS // 2)
                def _(t):
                    ra = 2 * t
                    rb = 2 * t + 1
                    gra = g * RPS + ra
                    grb = g * RPS + rb
                    g_wait(sv, ra, rows_a, gsem_a)

                    @pl.when(t > 0)
                    def _():
                        s_wait(grb - 2, rows_b, ssem_b)

                    g_start(sv, rb, rows_b, gsem_b)
                    compute(sv, ev, ra, gra, rows_a)
                    s_start(gra, rows_a, ssem_a)
                    g_wait(sv, rb, rows_b, gsem_b)
                    s_wait(gra, rows_a, ssem_a)

                    @pl.when(t < RPS // 2 - 1)
                    def _():
                        g_start(sv, ra + 2, rows_a, gsem_a)

                    compute(sv, ev, rb, grb, rows_b)
                    s_start(grb, rows_b, ssem_b)

                s_wait(RPS - 1 + g * RPS, rows_b, ssem_b)

        @pl.when(c == 0)
        def _():
            pass_b(h_hbm.at[0])

        @pl.when(c == 1)
        def _():
            pass_b(h_hbm.at[1])

        plsc.subcore_barrier()

        def copy_out(out_pl):
            @pl.loop(0, ROWS_W // CH)
            def _(j):
                r0 = pl.multiple_of(s * ROWS_W + j * CH, CH)
                pltpu.sync_copy(out_acc.at[pl.ds(r0, CH)],
                                out_pl.at[pl.ds(r0, CH)])

        @pl.when(c == 0)
        def _():
            copy_out(out_hbm.at[0])

        @pl.when(c == 1)
        def _():
            copy_out(out_hbm.at[1])

    return sc_edge


_sc_edge_hid = _make_sc_edge(D_HID)
_sc_edge_out = _make_sc_edge(D_OUT)


def kernel(x, edge_index, edge_weight, W1, a1_src, a1_dst, b1,
           W2, a2_src, a2_dst, b2):
    src = edge_index[0].astype(jnp.int32)
    dst = edge_index[1].astype(jnp.int32)
    dst2 = dst.reshape(NS, NCH, CH)
    ew = edge_weight.astype(jnp.float32)

    h1, av1, bv1, sh1 = _tc1(x, W1, a1_src, a1_dst)
    p1 = _sc_edge_hid(src, dst2, ew, av1, bv1, sh1, h1)
    h2, av2, bv2, sh2 = _tc2(p1, b1, W2, a2_src, a2_dst)
    p2 = _sc_edge_out(src, dst2, ew, av2, bv2, sh2, h2)
    return _tc3(p2, b2)


# speedup vs baseline: 37.4987x; 1.0316x over previous
import dataclasses
import functools

import jax
import jax.numpy as jnp
from jax import lax
from jax.experimental import pallas as pl
from jax.experimental.pallas import tpu as pltpu
from jax.experimental.pallas import tpu_sc as plsc

N_NODES = 10000
N_PAD = 10240
N_EDGES = 320000
D_IN = 128
D_HID = 128
D_OUT = 64

NC, NS, LANES = 2, 16, 16
E_T = N_EDGES // NS
CH = 80
NCH = E_T // CH
SUP = 4000
NSUP = E_T // SUP
RPS = SUP // CH
ROWS_W = N_PAD // NS


def _leaky(v):
    return jnp.where(v >= 0, v, 0.2 * v)


def _dense_tail(h, as_vec, ad_vec, d2, h_ref, av_ref, bv_ref, sh_ref):
    h_ref[0] = h[:, :d2]
    h_ref[1] = h[:, d2:]
    av = jnp.sum(h * as_vec[None, :], axis=1)
    bv = jnp.sum(h * ad_vec[None, :], axis=1)
    av_ref[...] = av
    bv_ref[...] = bv
    m = jnp.max(av) + jnp.max(bv)
    sh_ref[...] = jnp.full((LANES,), _leaky(m), jnp.float32)


def _tc1_body(x_ref, w_ref, as_ref, ad_ref, h_ref, av_ref, bv_ref, sh_ref):
    h = jnp.dot(x_ref[...], w_ref[...], preferred_element_type=jnp.float32)
    _dense_tail(h, as_ref[...], ad_ref[...], D_HID // 2,
                h_ref, av_ref, bv_ref, sh_ref)


def _tc2_body(p_ref, w_ref, as_ref, ad_ref, h_ref, av_ref, bv_ref,
              sh_ref):
    xx = jnp.concatenate(
        [p_ref[0, :N_NODES, :], p_ref[1, :N_NODES, :]], axis=1)
    xx = jnp.maximum(xx, 0.0)
    h = jnp.dot(xx, w_ref[...], preferred_element_type=jnp.float32)
    _dense_tail(h, as_ref[...], ad_ref[...], D_OUT // 2,
                h_ref, av_ref, bv_ref, sh_ref)


def _dense_out(d):
    return [
        jax.ShapeDtypeStruct((NC, N_NODES, d // 2), jnp.float32),
        jax.ShapeDtypeStruct((N_NODES,), jnp.float32),
        jax.ShapeDtypeStruct((N_NODES,), jnp.float32),
        jax.ShapeDtypeStruct((LANES,), jnp.float32),
    ]


_tc1 = pl.pallas_call(_tc1_body, out_shape=_dense_out(D_HID))
_tc2 = pl.pallas_call(_tc2_body, out_shape=_dense_out(D_OUT))


def _sc_compiler_params():
    cp = pltpu.CompilerParams(use_tc_tiling_on_sc=False)
    if "needs_layout_passes" in pltpu.CompilerParams.__dataclass_fields__:
        cp = dataclasses.replace(cp, needs_layout_passes=False)
    return cp


def _make_sc_edge(D):
    D2 = D // 2
    mesh = plsc.VectorSubcoreMesh(core_axis_name="c", subcore_axis_name="s")

    @functools.partial(
        pl.kernel,
        compiler_params=_sc_compiler_params(),
        out_type=jax.ShapeDtypeStruct((NC, N_PAD, D2), jnp.float32),
        mesh=mesh,
        scratch_types=[
            pltpu.VMEM((N_NODES,), jnp.float32),
            pltpu.VMEM((N_NODES,), jnp.float32),
            pltpu.VMEM((N_PAD,), jnp.float32),
            pltpu.VMEM((SUP,), jnp.int32),
            pltpu.VMEM((SUP,), jnp.float32),
            pltpu.VMEM((NS, ROWS_W // 4), jnp.float32),
            pltpu.VMEM((NC, D // 2), jnp.float32),
            pltpu.VMEM((NCH, CH), jnp.int32),
            pltpu.VMEM((CH, D // 2), jnp.float32),
            pltpu.VMEM((CH, D // 2), jnp.float32),
            pltpu.VMEM((CH,), jnp.float32),
            pltpu.VMEM((ROWS_W,), jnp.float32),
            pltpu.VMEM((LANES,), jnp.float32),
            pltpu.VMEM_SHARED((NS, N_PAD), jnp.float32),
            pltpu.VMEM_SHARED((N_PAD,), jnp.float32),
            pltpu.VMEM_SHARED((N_PAD, D2), jnp.float32),
            pltpu.SemaphoreType.DMA,
            pltpu.SemaphoreType.DMA,
            pltpu.SemaphoreType.DMA,
            pltpu.SemaphoreType.DMA,
            pltpu.SemaphoreType.DMA,
        ],
    )
    def sc_edge(src_hbm, dst2_hbm, ew_hbm, as_hbm, ad_hbm, sh_hbm, b_hbm,
                h_hbm, out_hbm,
                as_v, ad_v, denom_v, src_s, ew_s, red_v, bias_v,
                dst2_s, rows_a, rows_b,
                coeff_v, acc_v, shift_v, denom_stage, denom_fin,
                out_acc, gsem_a, gsem_b, ssem_a, ssem_b, stsem):
        c = lax.axis_index("c")
        s = lax.axis_index("s")
        zero16 = jnp.zeros((LANES,), jnp.float32)

        pltpu.sync_copy(as_hbm, as_v)
        pltpu.sync_copy(ad_hbm, ad_v)
        pltpu.sync_copy(sh_hbm, shift_v)
        pltpu.sync_copy(b_hbm, bias_v)
        pltpu.sync_copy(dst2_hbm.at[s], dst2_s)
        shift = shift_v[...]

        @pl.loop(0, N_PAD // LANES)
        def _(i):
            denom_v[pl.ds(i * LANES, LANES)] = zero16

        @pl.loop(0, CH)
        def _(e2):
            for k2 in range(D2 // LANES):
                sl2 = pl.ds(k2 * LANES, LANES)
                rows_a[e2, sl2] = bias_v[c, sl2]

        @pl.loop(0, ROWS_W // CH)
        def _(j):
            r0 = pl.multiple_of(s * ROWS_W + j * CH, CH)
            pltpu.sync_copy(rows_a, out_acc.at[pl.ds(r0, CH)])

        @pl.loop(0, NSUP)
        def _(g):
            base = pl.multiple_of(s * E_T + g * SUP, SUP)
            pltpu.sync_copy(src_hbm.at[pl.ds(base, SUP)], src_s)

            @pl.loop(0, RPS)
            def _(r2):
                gr = g * RPS + r2
                for i in range(CH // LANES):
                    ss = src_s[pl.ds(r2 * CH + i * LANES, LANES)]
                    dd = dst2_s[gr, pl.ds(i * LANES, LANES)]
                    e = (plsc.load_gather(as_v, [ss])
                         + plsc.load_gather(ad_v, [dd]))
                    ex = jnp.exp(_leaky(e) - shift)
                    plsc.addupdate_scatter(denom_v, [dd], ex)

        pltpu.sync_copy(denom_v, denom_stage.at[s])
        plsc.subcore_barrier()
        col = pl.multiple_of(s * ROWS_W, ROWS_W)
        HW = ROWS_W // 4
        for half in range(4):
            colh = pl.multiple_of(col + half * HW, HW)
            for w in range(NS):
                pltpu.async_copy(denom_stage.at[w, pl.ds(colh, HW)],
                                 red_v.at[w], stsem)
            for w in range(NS):
                pltpu.make_async_copy(denom_stage.at[w, pl.ds(colh, HW)],
                                      red_v.at[w], stsem).wait()

            @plsc.parallel_loop(0, HW // LANES, unroll=2)
            def _(i):
                sl = pl.ds(i * LANES, LANES)
                t0 = red_v[0, sl] + red_v[1, sl]
                t1 = red_v[2, sl] + red_v[3, sl]
                t2 = red_v[4, sl] + red_v[5, sl]
                t3 = red_v[6, sl] + red_v[7, sl]
                t4 = red_v[8, sl] + red_v[9, sl]
                t5 = red_v[10, sl] + red_v[11, sl]
                t6 = red_v[12, sl] + red_v[13, sl]
                t7 = red_v[14, sl] + red_v[15, sl]
                acc_v[pl.ds(half * HW + i * LANES, LANES)] = (
                    ((t0 + t1) + (t2 + t3)) + ((t4 + t5) + (t6 + t7)))

        pltpu.sync_copy(acc_v, denom_fin.at[pl.ds(col, ROWS_W)])
        plsc.subcore_barrier()
        pltpu.sync_copy(denom_fin, denom_v)

        def pass_b(h_pl):
            def g_start(sv, r2, buf, sem):
                pltpu.async_copy(
                    h_pl.at[sv.at[pl.ds(r2 * CH, CH)]], buf, sem)

            def g_wait(sv, r2, buf, sem):
                pltpu.make_async_copy(
                    h_pl.at[sv.at[pl.ds(r2 * CH, CH)]], buf, sem).wait()

            def s_start(gr, buf, sem):
                pltpu.async_copy(buf, out_acc.at[dst2_s.at[gr]], sem,
                                 add=True)

            def s_wait(gr, buf, sem):
                pltpu.make_async_copy(
                    buf, out_acc.at[dst2_s.at[gr]], sem).wait()

            def compute(sv, ev, r2, gr, buf):
                for i in range(CH // LANES):
                    sl = pl.ds(r2 * CH + i * LANES, LANES)
                    ss = sv[sl]
                    dd = dst2_s[gr, pl.ds(i * LANES, LANES)]
                    e = (plsc.load_gather(as_v, [ss])
                         + plsc.load_gather(ad_v, [dd]))
                    ex = jnp.exp(_leaky(e) - shift)
                    den = plsc.load_gather(denom_v, [dd])
                    coeff_v[pl.ds(i * LANES, LANES)] = (
                        ex * ev[sl] / (den + 1e-16))

                @plsc.parallel_loop(0, CH, unroll=4)
                def _(e2):
                    spl = plsc.load_gather(
                        coeff_v, [lax.broadcast(e2, (LANES,))])
                    for k2 in range(D2 // LANES):
                        sl2 = pl.ds(k2 * LANES, LANES)
                        buf[e2, sl2] = buf[e2, sl2] * spl

            def stage_start(g, sv, ev):
                base = pl.multiple_of(s * E_T + g * SUP, SUP)
                pltpu.async_copy(src_hbm.at[pl.ds(base, SUP)], sv, stsem)
                pltpu.async_copy(ew_hbm.at[pl.ds(base, SUP)], ev, stsem)

            def stage_wait(g, sv, ev):
                base = pl.multiple_of(s * E_T + g * SUP, SUP)
                pltpu.make_async_copy(
                    src_hbm.at[pl.ds(base, SUP)], sv, stsem).wait()
                pltpu.make_async_copy(
                    ew_hbm.at[pl.ds(base, SUP)], ev, stsem).wait()

            stage_start(0, src_s, ew_s)
            for g in range(NSUP):
                sv, ev = src_s, ew_s
                if g > 0:
                    stage_start(g, sv, ev)
                stage_wait(g, sv, ev)
                g_start(sv, 0, rows_a, gsem_a)

                @pl.loop(0, RPS // 2)
                def _(t):
                    ra = 2 * t
                    rb = 2 * t + 1
                    gra = g * RPS + ra
                    grb = g * RPS + rb
                    g_wait(sv, ra, rows_a, gsem_a)

                    @pl.when(t > 0)
                    def _():
                        s_wait(grb - 2, rows_b, ssem_b)

                    g_start(sv, rb, rows_b, gsem_b)
                    compute(sv, ev, ra, gra, rows_a)
                    s_start(gra, rows_a, ssem_a)
                    g_wait(sv, rb, rows_b, gsem_b)
                    s_wait(gra, rows_a, ssem_a)

                    @pl.when(t < RPS // 2 - 1)
                    def _():
                        g_start(sv, ra + 2, rows_a, gsem_a)

                    compute(sv, ev, rb, grb, rows_b)
                    s_start(grb, rows_b, ssem_b)

                s_wait(RPS - 1 + g * RPS, rows_b, ssem_b)

        @pl.when(c == 0)
        def _():
            pass_b(h_hbm.at[0])

        @pl.when(c == 1)
        def _():
            pass_b(h_hbm.at[1])

        plsc.subcore_barrier()

        def copy_out(out_pl):
            @pl.loop(0, ROWS_W // CH)
            def _(j):
                r0 = pl.multiple_of(s * ROWS_W + j * CH, CH)
                pltpu.sync_copy(out_acc.at[pl.ds(r0, CH)],
                                out_pl.at[pl.ds(r0, CH)])

        @pl.when(c == 0)
        def _():
            copy_out(out_hbm.at[0])

        @pl.when(c == 1)
        def _():
            copy_out(out_hbm.at[1])

    return sc_edge


_sc_edge_hid = _make_sc_edge(D_HID)
_sc_edge_out = _make_sc_edge(D_OUT)


def kernel(x, edge_index, edge_weight, W1, a1_src, a1_dst, b1,
           W2, a2_src, a2_dst, b2):
    src = edge_index[0].astype(jnp.int32)
    dst = edge_index[1].astype(jnp.int32)
    dst2 = dst.reshape(NS, NCH, CH)
    ew = edge_weight.astype(jnp.float32)

    h1, av1, bv1, sh1 = _tc1(x, W1, a1_src, a1_dst)
    p1 = _sc_edge_hid(src, dst2, ew, av1, bv1, sh1,
                      b1.reshape(NC, D_HID // 2), h1)
    h2, av2, bv2, sh2 = _tc2(p1, W2, a2_src, a2_dst)
    p2 = _sc_edge_out(src, dst2, ew, av2, bv2, sh2,
                      b2.reshape(NC, D_OUT // 2), h2)
    return jnp.concatenate([p2[0, :N_NODES], p2[1, :N_NODES]], axis=1)
